# Initial kernel scaffold; baseline (speedup 1.0000x reference)
import jax
import jax.numpy as jnp
from jax.experimental import pallas as pl


def kernel(x, emb):
    raise NotImplementedError("write your pallas kernel here")


# speedup vs baseline: 2.7294x; 2.7294x over previous
import functools

import jax
import jax.numpy as jnp
import numpy as np
from jax import lax
from jax.experimental import pallas as pl
from jax.experimental.pallas import tpu as pltpu
from jax.experimental.pallas import tpu_sc as plsc

D = 64
S = 200
B = 1024

NC, NS, L = 2, 16, 16
NW = NC * NS
BROWS_PER_W = B // NW
HALF = S // 2


def _pe_table() -> np.ndarray:
    position = np.arange(S)[:, np.newaxis]
    div_term = np.exp(np.arange(0, D, 2) * (-np.log(10000.0) / D))
    pe = np.zeros((S, D), dtype=np.float32)
    pe[:, 0::2] = np.sin(position * div_term)
    pe[:, 1::2] = np.cos(position * div_term)
    return pe


_PE = _pe_table()

_mesh = plsc.VectorSubcoreMesh(core_axis_name="c", subcore_axis_name="s")


@functools.partial(
    pl.kernel,
    out_type=jax.ShapeDtypeStruct((B * S, D), jnp.float32),
    mesh=_mesh,
    scratch_types=[
        pltpu.VMEM((BROWS_PER_W, 2, HALF), jnp.int32),
        pltpu.VMEM((S, D), jnp.float32),
        pltpu.VMEM((S, D), jnp.float32),
        pltpu.SemaphoreType.DMA,
    ],
    compiler_params=pltpu.CompilerParams(use_tc_tiling_on_sc=False),
)
def _encode(emb_hbm, x_hbm, pe_hbm, out_hbm, idx_v, pe_v, rows_v, sem):
    wid = lax.axis_index("s") * NC + lax.axis_index("c")
    base = wid * BROWS_PER_W * S
    pltpu.sync_copy(x_hbm.at[wid], idx_v)
    pltpu.sync_copy(pe_hbm, pe_v)

    def batch_row(r, _):
        pltpu.async_copy(
            emb_hbm.at[idx_v.at[r, 0]], rows_v.at[pl.ds(0, HALF)], sem)
        pltpu.async_copy(
            emb_hbm.at[idx_v.at[r, 1]], rows_v.at[pl.ds(HALF, HALF)], sem)
        pltpu.make_async_copy(
            emb_hbm.at[idx_v.at[r, 0]], rows_v.at[pl.ds(0, HALF)], sem).wait()
        pltpu.make_async_copy(
            emb_hbm.at[idx_v.at[r, 1]], rows_v.at[pl.ds(HALF, HALF)], sem).wait()

        def add_row(i, _):
            for q in range(D // L):
                p = pe_v[i, pl.ds(q * L, L)]
                plsc.addupdate(rows_v.at[i, pl.ds(q * L, L)], p)
            return 0

        lax.fori_loop(0, S, add_row, 0)
        pltpu.sync_copy(rows_v, out_hbm.at[pl.ds(base + r * S, S)])
        return 0

    lax.fori_loop(0, BROWS_PER_W, batch_row, 0)


def kernel(x, emb):
    xw = x.reshape(NW, BROWS_PER_W, 2, HALF)
    out = _encode(emb, xw, jnp.asarray(_PE))
    return out.reshape(B, S, D)


# speedup vs baseline: 3.1080x; 1.1387x over previous
import functools

import jax
import jax.numpy as jnp
import numpy as np
from jax import lax
from jax.experimental import pallas as pl
from jax.experimental.pallas import tpu as pltpu
from jax.experimental.pallas import tpu_sc as plsc

D = 64
S = 200
B = 1024

NC, NS, L = 2, 16, 16
NW = NC * NS
BROWS_PER_W = B // NW
HALF = S // 2


def _pe_table() -> np.ndarray:
    position = np.arange(S)[:, np.newaxis]
    div_term = np.exp(np.arange(0, D, 2) * (-np.log(10000.0) / D))
    pe = np.zeros((S, D), dtype=np.float32)
    pe[:, 0::2] = np.sin(position * div_term)
    pe[:, 1::2] = np.cos(position * div_term)
    return pe


_PE = _pe_table()

_mesh = plsc.VectorSubcoreMesh(core_axis_name="c", subcore_axis_name="s")


@functools.partial(
    pl.kernel,
    out_type=jax.ShapeDtypeStruct((B * S, D), jnp.float32),
    mesh=_mesh,
    scratch_types=[
        pltpu.VMEM((BROWS_PER_W, 2, HALF), jnp.int32),
        pltpu.VMEM((S, D), jnp.float32),
        pltpu.VMEM((2, S, D), jnp.float32),
        pltpu.SemaphoreType.DMA,
        pltpu.SemaphoreType.DMA,
    ],
    compiler_params=pltpu.CompilerParams(use_tc_tiling_on_sc=False),
)
def _encode(emb_hbm, x_hbm, pe_hbm, out_hbm, idx_v, pe_v, rows_v, sem0, sem1):
    wid = lax.axis_index("s") * NC + lax.axis_index("c")
    base = wid * BROWS_PER_W * S
    pltpu.sync_copy(x_hbm.at[wid], idx_v)
    pltpu.sync_copy(pe_hbm, pe_v)

    sems = (sem0, sem1)

    def issue_gather(r, b):
        pltpu.async_copy(
            emb_hbm.at[idx_v.at[r, 0]], rows_v.at[b, pl.ds(0, HALF)], sems[b])
        pltpu.async_copy(
            emb_hbm.at[idx_v.at[r, 1]], rows_v.at[b, pl.ds(HALF, HALF)],
            sems[b])

    def wait_gather(r, b):
        pltpu.make_async_copy(
            emb_hbm.at[idx_v.at[r, 0]], rows_v.at[b, pl.ds(0, HALF)],
            sems[b]).wait()
        pltpu.make_async_copy(
            emb_hbm.at[idx_v.at[r, 1]], rows_v.at[b, pl.ds(HALF, HALF)],
            sems[b]).wait()

    for b in range(2):
        issue_gather(b, b)

    def row_pair(g, _):
        for b in range(2):
            r = 2 * g + b
            buf = rows_v.at[b]
            wait_gather(r, b)

            @plsc.parallel_loop(0, S, step=1, unroll=4)
            def _add(i):
                for q in range(D // L):
                    p = pe_v[i, pl.ds(q * L, L)]
                    plsc.addupdate(buf.at[i, pl.ds(q * L, L)], p)

            pltpu.sync_copy(buf, out_hbm.at[pl.ds(base + r * S, S)])

            @pl.when(r + 2 < BROWS_PER_W)
            def _():
                issue_gather(r + 2, b)
        return 0

    lax.fori_loop(0, BROWS_PER_W // 2, row_pair, 0)


def kernel(x, emb):
    xw = x.reshape(NW, BROWS_PER_W, 2, HALF)
    out = _encode(emb, xw, jnp.asarray(_PE))
    return out.reshape(B, S, D)
